# baseline (speedup 1.0000x reference)
import functools

import jax
import jax.numpy as jnp
import numpy as np
from jax import lax
from jax.experimental import pallas as pl
from jax.experimental.pallas import tpu as pltpu
from jax.experimental.pallas import tpu_sc as plsc

N_NODES = 10000
N_EDGES = 160000
C = 16
HID = 256

NW = 32
CBS = 128
NCHUNK = 40
EPW = 5000
EPWP = CBS * NCHUNK
DUMMY = N_NODES
ZR = 632
NP = ZR * 16

_SC_PARAMS = pltpu.CompilerParams(use_tc_tiling_on_sc=False)


def _sc_mesh():
    return plsc.VectorSubcoreMesh(core_axis_name="c", subcore_axis_name="s")


_R_NP = np.zeros((C, HID), np.float32)
_R_NP[np.arange(HID) // C, np.arange(HID)] = 1.0
_S_NP = np.zeros((HID, C), np.float32)
_S_NP[np.arange(HID), np.arange(HID) % C] = 1.0


@functools.lru_cache(maxsize=None)
def _make_sc_gather():
    @functools.partial(
        pl.kernel,
        mesh=_sc_mesh(),
        out_type=jax.ShapeDtypeStruct((N_EDGES, 128), jnp.float32),
        scratch_types=[
            pltpu.VMEM((NCHUNK, CBS), jnp.int32),
            pltpu.VMEM((EPWP, C), jnp.float32),
            pltpu.SemaphoreType.DMA,
        ],
        compiler_params=_SC_PARAMS,
    )
    def _sc_gather(table_hbm, idx_hbm, out_hbm, idx_v, rows_v, sem):
        cid = lax.axis_index("c")
        sid = lax.axis_index("s")
        wid = sid * 2 + cid
        pltpu.sync_copy(idx_hbm.at[wid], idx_v)

        def fire(j, carry):
            pltpu.async_copy(
                table_hbm.at[idx_v.at[j]],
                rows_v.at[pl.ds(j * CBS, CBS)],
                sem,
            )
            return carry

        lax.fori_loop(0, NCHUNK, fire, 0)

        def drain(j, carry):
            pltpu.make_async_copy(
                table_hbm.at[idx_v.at[j]],
                rows_v.at[pl.ds(j * CBS, CBS)],
                sem,
            ).wait()
            return carry

        lax.fori_loop(0, NCHUNK, drain, 0)
        pltpu.sync_copy(rows_v.at[pl.ds(0, EPW)],
                        out_hbm.at[pl.ds(wid * EPW, EPW), pl.ds(0, C)])

    return _sc_gather


def _scatter_body(with_cnt, msg_hbm, dst_hbm, sum_out, cnt_out,
                  msg_v, idx_v, ones_v, zrow_v, sum_s, cnt_s, sem):
    cid = lax.axis_index("c")
    sid = lax.axis_index("s")
    wid = sid * 2 + cid

    pltpu.sync_copy(msg_hbm.at[pl.ds(wid * EPW, EPW), pl.ds(0, C)],
                    msg_v.at[pl.ds(0, EPW)])
    pltpu.sync_copy(dst_hbm.at[wid], idx_v)

    def zfill(r, carry):
        zrow_v[r] = jnp.zeros((C,), jnp.float32)
        return carry

    lax.fori_loop(0, ZR, zfill, 0)

    def zpad(r, carry):
        msg_v[EPW + r] = jnp.zeros((C,), jnp.float32)
        return carry

    lax.fori_loop(0, EPWP - EPW, zpad, 0)
    pltpu.sync_copy(zrow_v, sum_s.at[pl.ds(sid * ZR, ZR)])
    if with_cnt:
        def ofill(r, carry):
            ones_v[r] = jnp.ones((C,), jnp.float32)
            return carry

        lax.fori_loop(0, CBS, ofill, 0)
        pltpu.sync_copy(zrow_v, cnt_s.at[pl.ds(sid * ZR, ZR)])

    plsc.subcore_barrier()

    def scat(j, carry):
        pltpu.async_copy(msg_v.at[pl.ds(j * CBS, CBS)],
                         sum_s.at[idx_v.at[j]], sem, add=True)
        if with_cnt:
            pltpu.async_copy(ones_v, cnt_s.at[idx_v.at[j]], sem, add=True)
        return carry

    lax.fori_loop(0, NCHUNK, scat, 0)

    def scat_drain(j, carry):
        pltpu.make_async_copy(msg_v.at[pl.ds(j * CBS, CBS)],
                              sum_s.at[idx_v.at[j]], sem).wait()
        if with_cnt:
            pltpu.make_async_copy(ones_v, cnt_s.at[idx_v.at[j]], sem).wait()
        return carry

    lax.fori_loop(0, NCHUNK, scat_drain, 0)
    plsc.subcore_barrier()

    sl = pl.ds(sid * ZR, ZR)
    pltpu.sync_copy(sum_s.at[sl], sum_out.at[cid].at[sl])
    if with_cnt:
        pltpu.sync_copy(cnt_s.at[sl], cnt_out.at[cid].at[sl])


@functools.lru_cache(maxsize=None)
def _make_sc_scatter_cnt():
    @functools.partial(
        pl.kernel,
        mesh=_sc_mesh(),
        out_type=(
            jax.ShapeDtypeStruct((2, NP, C), jnp.float32),
            jax.ShapeDtypeStruct((2, NP, C), jnp.float32),
        ),
        scratch_types=[
            pltpu.VMEM((EPWP, C), jnp.float32),
            pltpu.VMEM((NCHUNK, CBS), jnp.int32),
            pltpu.VMEM((CBS, C), jnp.float32),
            pltpu.VMEM((ZR, C), jnp.float32),
            pltpu.VMEM_SHARED((NP, C), jnp.float32),
            pltpu.VMEM_SHARED((NP, C), jnp.float32),
            pltpu.SemaphoreType.DMA,
        ],
        compiler_params=_SC_PARAMS,
    )
    def _sc_scatter_cnt(msg_hbm, dst_hbm, sum_out, cnt_out,
                        msg_v, idx_v, ones_v, zrow_v, sum_s, cnt_s, sem):
        _scatter_body(True, msg_hbm, dst_hbm, sum_out, cnt_out,
                      msg_v, idx_v, ones_v, zrow_v, sum_s, cnt_s, sem)

    return _sc_scatter_cnt


@functools.lru_cache(maxsize=None)
def _make_sc_scatter():
    @functools.partial(
        pl.kernel,
        mesh=_sc_mesh(),
        out_type=jax.ShapeDtypeStruct((2, NP, C), jnp.float32),
        scratch_types=[
            pltpu.VMEM((EPWP, C), jnp.float32),
            pltpu.VMEM((NCHUNK, CBS), jnp.int32),
            pltpu.VMEM((ZR, C), jnp.float32),
            pltpu.VMEM_SHARED((NP, C), jnp.float32),
            pltpu.SemaphoreType.DMA,
        ],
        compiler_params=_SC_PARAMS,
    )
    def _sc_scatter(msg_hbm, dst_hbm, sum_out,
                    msg_v, idx_v, zrow_v, sum_s, sem):
        _scatter_body(False, msg_hbm, dst_hbm, sum_out, None,
                      msg_v, idx_v, None, zrow_v, sum_s, None, sem)

    return _sc_scatter


EB = 5000


def _msg_body(ea_ref, xs_ref, w1_ref, b1_ref, w2_ref, b2_ref,
              r_ref, s_ref, msg_ref):
    g = jnp.dot(ea_ref[...], w1_ref[...],
                preferred_element_type=jnp.float32) + b1_ref[...]
    g = jnp.where(g > 0, g, 0.01 * g)
    h = jnp.dot(g.astype(jnp.bfloat16), w2_ref[...],
                preferred_element_type=jnp.float32) + b2_ref[...]
    xs = xs_ref[...][:, :C]
    xe = jnp.dot(xs, r_ref[...], preferred_element_type=jnp.float32)
    msg = jnp.dot((xe * h).astype(jnp.bfloat16), s_ref[...],
                  preferred_element_type=jnp.float32)
    msg_ref[...] = jnp.concatenate(
        [msg, jnp.zeros((EB, 128 - C), jnp.float32)], axis=1)


def _tc_msg(ea, xs_packed, w1, b1, w2, b2):
    r = jnp.asarray(_R_NP)
    s = jnp.asarray(_S_NP, dtype=jnp.bfloat16)
    grid = N_EDGES // EB
    return pl.pallas_call(
        _msg_body,
        grid=(grid,),
        in_specs=[
            pl.BlockSpec((EB, C), lambda i: (i, 0)),
            pl.BlockSpec((EB, 128), lambda i: (i, 0)),
            pl.BlockSpec((C, HID), lambda i: (0, 0)),
            pl.BlockSpec((1, HID), lambda i: (0, 0)),
            pl.BlockSpec((HID, HID), lambda i: (0, 0)),
            pl.BlockSpec((1, HID), lambda i: (0, 0)),
            pl.BlockSpec((C, HID), lambda i: (0, 0)),
            pl.BlockSpec((HID, C), lambda i: (0, 0)),
        ],
        out_specs=pl.BlockSpec((EB, 128), lambda i: (i, 0)),
        out_shape=jax.ShapeDtypeStruct((N_EDGES, 128), jnp.float32),
        compiler_params=pltpu.CompilerParams(
            dimension_semantics=("arbitrary",)),
    )(ea, xs_packed, w1, b1.reshape(1, HID),
      w2.astype(jnp.bfloat16), b2.reshape(1, HID), r, s)


def _epi0_body(sum_ref, cnt_ref, x_ref, root_ref, bias_ref,
               h_ref, recip_ref):
    cnt = cnt_ref[0, :N_NODES, :] + cnt_ref[1, :N_NODES, :]
    recip = 1.0 / jnp.maximum(cnt, 1.0)
    summ = sum_ref[0, :N_NODES, :] + sum_ref[1, :N_NODES, :]
    val = summ * recip + jnp.dot(x_ref[...], root_ref[...],
                                 preferred_element_type=jnp.float32) \
        + bias_ref[...]
    h_ref[...] = jnp.where(val > 0, val, 0.01 * val)
    recip_ref[...] = recip


def _tc_epi0(sums, cnts, x, root, bias):
    return pl.pallas_call(
        _epi0_body,
        in_specs=[
            pl.BlockSpec((2, NP, C), lambda: (0, 0, 0)),
            pl.BlockSpec((2, NP, C), lambda: (0, 0, 0)),
            pl.BlockSpec((N_NODES, C), lambda: (0, 0)),
            pl.BlockSpec((C, C), lambda: (0, 0)),
            pl.BlockSpec((1, C), lambda: (0, 0)),
        ],
        out_specs=(
            pl.BlockSpec((N_NODES, C), lambda: (0, 0)),
            pl.BlockSpec((N_NODES, C), lambda: (0, 0)),
        ),
        out_shape=(
            jax.ShapeDtypeStruct((N_NODES, C), jnp.float32),
            jax.ShapeDtypeStruct((N_NODES, C), jnp.float32),
        ),
    )(sums, cnts, x, root, bias.reshape(1, C))


def _epi1_body(sum_ref, recip_ref, h_ref, root_ref, bias_ref, out_ref):
    summ = sum_ref[0, :N_NODES, :] + sum_ref[1, :N_NODES, :]
    out_ref[...] = summ * recip_ref[...] \
        + jnp.dot(h_ref[...], root_ref[...],
                  preferred_element_type=jnp.float32) + bias_ref[...]


def _tc_epi1(sums, recip, h, root, bias):
    return pl.pallas_call(
        _epi1_body,
        in_specs=[
            pl.BlockSpec((2, NP, C), lambda: (0, 0, 0)),
            pl.BlockSpec((N_NODES, C), lambda: (0, 0)),
            pl.BlockSpec((N_NODES, C), lambda: (0, 0)),
            pl.BlockSpec((C, C), lambda: (0, 0)),
            pl.BlockSpec((1, C), lambda: (0, 0)),
        ],
        out_specs=pl.BlockSpec((N_NODES, C), lambda: (0, 0)),
        out_shape=jax.ShapeDtypeStruct((N_NODES, C), jnp.float32),
    )(sums, recip, h, root, bias.reshape(1, C))


def kernel(x, edge_index, edge_attr, W1_0, b1_0, W2_0, b2_0, root_0, bias_0,
           W1_1, b1_1, W2_1, b2_1, root_1, bias_1):
    padw = ((0, 0), (0, EPWP - EPW))
    src3 = jnp.pad(edge_index[0].reshape(NW, EPW), padw)
    src3 = src3.reshape(NW, NCHUNK, CBS)
    dst3 = jnp.pad(edge_index[1].reshape(NW, EPW), padw,
                   constant_values=DUMMY)
    dst3 = dst3.reshape(NW, NCHUNK, CBS)

    gather = _make_sc_gather()
    xs0 = gather(x, src3)
    msg0 = _tc_msg(edge_attr, xs0, W1_0, b1_0, W2_0, b2_0)
    sum0, cnt = _make_sc_scatter_cnt()(msg0, dst3)
    h1, recip = _tc_epi0(sum0, cnt, x, root_0, bias_0)

    xs1 = gather(h1, src3)
    msg1 = _tc_msg(edge_attr, xs1, W1_1, b1_1, W2_1, b2_1)
    sum1 = _make_sc_scatter()(msg1, dst3)
    return _tc_epi1(sum1, recip, h1, root_1, bias_1)


# speedup vs baseline: 1.0216x; 1.0216x over previous
import functools

import jax
import jax.numpy as jnp
import numpy as np
from jax import lax
from jax.experimental import pallas as pl
from jax.experimental.pallas import tpu as pltpu
from jax.experimental.pallas import tpu_sc as plsc

N_NODES = 10000
N_EDGES = 160000
C = 16
HID = 256

NW = 32
CBS = 128
NCHUNK = 40
EPW = 5000
EPWP = CBS * NCHUNK
DUMMY = N_NODES
ZR = 632
NP = ZR * 16

_SC_PARAMS = pltpu.CompilerParams(use_tc_tiling_on_sc=False)


def _sc_mesh():
    return plsc.VectorSubcoreMesh(core_axis_name="c", subcore_axis_name="s")


_R_NP = np.zeros((C, HID), np.float32)
_R_NP[np.arange(HID) // C, np.arange(HID)] = 1.0
_S_NP = np.zeros((HID, C), np.float32)
_S_NP[np.arange(HID), np.arange(HID) % C] = 1.0


@functools.lru_cache(maxsize=None)
def _make_sc_gather():
    @functools.partial(
        pl.kernel,
        mesh=_sc_mesh(),
        out_type=jax.ShapeDtypeStruct((N_EDGES, 128), jnp.float32),
        scratch_types=[
            pltpu.VMEM((NCHUNK, CBS), jnp.int32),
            pltpu.VMEM((EPWP, C), jnp.float32),
            pltpu.SemaphoreType.DMA,
        ],
        compiler_params=_SC_PARAMS,
    )
    def _sc_gather(table_hbm, idx_hbm, out_hbm, idx_v, rows_v, sem):
        cid = lax.axis_index("c")
        sid = lax.axis_index("s")
        wid = sid * 2 + cid
        pltpu.sync_copy(idx_hbm.at[wid], idx_v)

        def fire(j, carry):
            pltpu.async_copy(
                table_hbm.at[idx_v.at[j]],
                rows_v.at[pl.ds(j * CBS, CBS)],
                sem,
            )
            return carry

        lax.fori_loop(0, NCHUNK, fire, 0)

        def drain(j, carry):
            pltpu.make_async_copy(
                table_hbm.at[idx_v.at[j]],
                rows_v.at[pl.ds(j * CBS, CBS)],
                sem,
            ).wait()
            return carry

        lax.fori_loop(0, NCHUNK, drain, 0)
        pltpu.sync_copy(rows_v.at[pl.ds(0, EPW)],
                        out_hbm.at[pl.ds(wid * EPW, EPW), pl.ds(0, C)])

    return _sc_gather


def _scatter_body(with_cnt, msg_hbm, dst_hbm, sum_out, cnt_out,
                  msg_v, idx_v, ones_v, zrow_v, sum_s, cnt_s, sem):
    cid = lax.axis_index("c")
    sid = lax.axis_index("s")
    wid = sid * 2 + cid

    pltpu.sync_copy(msg_hbm.at[pl.ds(wid * EPW, EPW), pl.ds(0, C)],
                    msg_v.at[pl.ds(0, EPW)])
    pltpu.sync_copy(dst_hbm.at[wid], idx_v)

    def zfill(r, carry):
        zrow_v[r] = jnp.zeros((C,), jnp.float32)
        return carry

    lax.fori_loop(0, ZR, zfill, 0)

    def zpad(r, carry):
        msg_v[EPW + r] = jnp.zeros((C,), jnp.float32)
        return carry

    lax.fori_loop(0, EPWP - EPW, zpad, 0)
    pltpu.sync_copy(zrow_v, sum_s.at[pl.ds(sid * ZR, ZR)])
    if with_cnt:
        def ofill(r, carry):
            ones_v[r] = jnp.ones((C,), jnp.float32)
            return carry

        lax.fori_loop(0, CBS, ofill, 0)
        pltpu.sync_copy(zrow_v, cnt_s.at[pl.ds(sid * ZR, ZR)])

    plsc.subcore_barrier()

    def scat(j, carry):
        pltpu.async_copy(msg_v.at[pl.ds(j * CBS, CBS)],
                         sum_s.at[idx_v.at[j]], sem, add=True)
        if with_cnt:
            pltpu.async_copy(ones_v, cnt_s.at[idx_v.at[j]], sem, add=True)
        return carry

    lax.fori_loop(0, NCHUNK, scat, 0)

    def scat_drain(j, carry):
        pltpu.make_async_copy(msg_v.at[pl.ds(j * CBS, CBS)],
                              sum_s.at[idx_v.at[j]], sem).wait()
        if with_cnt:
            pltpu.make_async_copy(ones_v, cnt_s.at[idx_v.at[j]], sem).wait()
        return carry

    lax.fori_loop(0, NCHUNK, scat_drain, 0)
    plsc.subcore_barrier()

    sl = pl.ds(sid * ZR, ZR)
    pltpu.sync_copy(sum_s.at[sl], sum_out.at[cid].at[sl, pl.ds(0, C)])
    if with_cnt:
        pltpu.sync_copy(cnt_s.at[sl], cnt_out.at[cid].at[sl, pl.ds(0, C)])


@functools.lru_cache(maxsize=None)
def _make_sc_scatter_cnt():
    @functools.partial(
        pl.kernel,
        mesh=_sc_mesh(),
        out_type=(
            jax.ShapeDtypeStruct((2, NP, 128), jnp.float32),
            jax.ShapeDtypeStruct((2, NP, 128), jnp.float32),
        ),
        scratch_types=[
            pltpu.VMEM((EPWP, C), jnp.float32),
            pltpu.VMEM((NCHUNK, CBS), jnp.int32),
            pltpu.VMEM((CBS, C), jnp.float32),
            pltpu.VMEM((ZR, C), jnp.float32),
            pltpu.VMEM_SHARED((NP, C), jnp.float32),
            pltpu.VMEM_SHARED((NP, C), jnp.float32),
            pltpu.SemaphoreType.DMA,
        ],
        compiler_params=_SC_PARAMS,
    )
    def _sc_scatter_cnt(msg_hbm, dst_hbm, sum_out, cnt_out,
                        msg_v, idx_v, ones_v, zrow_v, sum_s, cnt_s, sem):
        _scatter_body(True, msg_hbm, dst_hbm, sum_out, cnt_out,
                      msg_v, idx_v, ones_v, zrow_v, sum_s, cnt_s, sem)

    return _sc_scatter_cnt


@functools.lru_cache(maxsize=None)
def _make_sc_scatter():
    @functools.partial(
        pl.kernel,
        mesh=_sc_mesh(),
        out_type=jax.ShapeDtypeStruct((2, NP, 128), jnp.float32),
        scratch_types=[
            pltpu.VMEM((EPWP, C), jnp.float32),
            pltpu.VMEM((NCHUNK, CBS), jnp.int32),
            pltpu.VMEM((ZR, C), jnp.float32),
            pltpu.VMEM_SHARED((NP, C), jnp.float32),
            pltpu.SemaphoreType.DMA,
        ],
        compiler_params=_SC_PARAMS,
    )
    def _sc_scatter(msg_hbm, dst_hbm, sum_out,
                    msg_v, idx_v, zrow_v, sum_s, sem):
        _scatter_body(False, msg_hbm, dst_hbm, sum_out, None,
                      msg_v, idx_v, None, zrow_v, sum_s, None, sem)

    return _sc_scatter


EB = 5000


def _msg_body(ea_ref, xs_ref, w1_ref, b1_ref, w2_ref, b2_ref,
              r_ref, s_ref, msg_ref):
    g = jnp.dot(ea_ref[...], w1_ref[...],
                preferred_element_type=jnp.float32) + b1_ref[...]
    g = jnp.where(g > 0, g, 0.01 * g)
    h = jnp.dot(g.astype(jnp.bfloat16), w2_ref[...],
                preferred_element_type=jnp.float32) + b2_ref[...]
    xs = xs_ref[...][:, :C]
    xe = jnp.dot(xs, r_ref[...], preferred_element_type=jnp.float32)
    msg = jnp.dot((xe * h).astype(jnp.bfloat16), s_ref[...],
                  preferred_element_type=jnp.float32)
    msg_ref[...] = jnp.concatenate(
        [msg, jnp.zeros((EB, 128 - C), jnp.float32)], axis=1)


def _tc_msg(ea, xs_packed, w1, b1, w2, b2):
    r = jnp.asarray(_R_NP)
    s = jnp.asarray(_S_NP, dtype=jnp.bfloat16)
    grid = N_EDGES // EB
    return pl.pallas_call(
        _msg_body,
        grid=(grid,),
        in_specs=[
            pl.BlockSpec((EB, C), lambda i: (i, 0)),
            pl.BlockSpec((EB, 128), lambda i: (i, 0)),
            pl.BlockSpec((C, HID), lambda i: (0, 0)),
            pl.BlockSpec((1, HID), lambda i: (0, 0)),
            pl.BlockSpec((HID, HID), lambda i: (0, 0)),
            pl.BlockSpec((1, HID), lambda i: (0, 0)),
            pl.BlockSpec((C, HID), lambda i: (0, 0)),
            pl.BlockSpec((HID, C), lambda i: (0, 0)),
        ],
        out_specs=pl.BlockSpec((EB, 128), lambda i: (i, 0)),
        out_shape=jax.ShapeDtypeStruct((N_EDGES, 128), jnp.float32),
        compiler_params=pltpu.CompilerParams(
            dimension_semantics=("arbitrary",)),
    )(ea, xs_packed, w1, b1.reshape(1, HID),
      w2.astype(jnp.bfloat16), b2.reshape(1, HID), r, s)


def _epi0_body(sum_ref, cnt_ref, x_ref, root_ref, bias_ref,
               h_ref, recip_ref):
    cnt = cnt_ref[0, :N_NODES, :C] + cnt_ref[1, :N_NODES, :C]
    recip = 1.0 / jnp.maximum(cnt, 1.0)
    summ = sum_ref[0, :N_NODES, :C] + sum_ref[1, :N_NODES, :C]
    val = summ * recip + jnp.dot(x_ref[...], root_ref[...],
                                 preferred_element_type=jnp.float32) \
        + bias_ref[...]
    h_ref[...] = jnp.where(val > 0, val, 0.01 * val)
    recip_ref[...] = recip


def _tc_epi0(sums, cnts, x, root, bias):
    return pl.pallas_call(
        _epi0_body,
        in_specs=[
            pl.BlockSpec((2, NP, 128), lambda: (0, 0, 0)),
            pl.BlockSpec((2, NP, 128), lambda: (0, 0, 0)),
            pl.BlockSpec((N_NODES, C), lambda: (0, 0)),
            pl.BlockSpec((C, C), lambda: (0, 0)),
            pl.BlockSpec((1, C), lambda: (0, 0)),
        ],
        out_specs=(
            pl.BlockSpec((N_NODES, C), lambda: (0, 0)),
            pl.BlockSpec((N_NODES, C), lambda: (0, 0)),
        ),
        out_shape=(
            jax.ShapeDtypeStruct((N_NODES, C), jnp.float32),
            jax.ShapeDtypeStruct((N_NODES, C), jnp.float32),
        ),
    )(sums, cnts, x, root, bias.reshape(1, C))


def _epi1_body(sum_ref, recip_ref, h_ref, root_ref, bias_ref, out_ref):
    summ = sum_ref[0, :N_NODES, :C] + sum_ref[1, :N_NODES, :C]
    out_ref[...] = summ * recip_ref[...] \
        + jnp.dot(h_ref[...], root_ref[...],
                  preferred_element_type=jnp.float32) + bias_ref[...]


def _tc_epi1(sums, recip, h, root, bias):
    return pl.pallas_call(
        _epi1_body,
        in_specs=[
            pl.BlockSpec((2, NP, 128), lambda: (0, 0, 0)),
            pl.BlockSpec((N_NODES, C), lambda: (0, 0)),
            pl.BlockSpec((N_NODES, C), lambda: (0, 0)),
            pl.BlockSpec((C, C), lambda: (0, 0)),
            pl.BlockSpec((1, C), lambda: (0, 0)),
        ],
        out_specs=pl.BlockSpec((N_NODES, C), lambda: (0, 0)),
        out_shape=jax.ShapeDtypeStruct((N_NODES, C), jnp.float32),
    )(sums, recip, h, root, bias.reshape(1, C))


def kernel(x, edge_index, edge_attr, W1_0, b1_0, W2_0, b2_0, root_0, bias_0,
           W1_1, b1_1, W2_1, b2_1, root_1, bias_1):
    padw = ((0, 0), (0, EPWP - EPW))
    src3 = jnp.pad(edge_index[0].reshape(NW, EPW), padw)
    src3 = src3.reshape(NW, NCHUNK, CBS)
    dst3 = jnp.pad(edge_index[1].reshape(NW, EPW), padw,
                   constant_values=DUMMY)
    dst3 = dst3.reshape(NW, NCHUNK, CBS)

    gather = _make_sc_gather()
    xs0 = gather(x, src3)
    msg0 = _tc_msg(edge_attr, xs0, W1_0, b1_0, W2_0, b2_0)
    sum0, cnt = _make_sc_scatter_cnt()(msg0, dst3)
    h1, recip = _tc_epi0(sum0, cnt, x, root_0, bias_0)

    xs1 = gather(h1, src3)
    msg1 = _tc_msg(edge_attr, xs1, W1_1, b1_1, W2_1, b2_1)
    sum1 = _make_sc_scatter()(msg1, dst3)
    return _tc_epi1(sum1, recip, h1, root_1, bias_1)
